# baseline (speedup 1.0000x reference)
import functools

import jax
import jax.numpy as jnp
from jax import lax
from jax.experimental import pallas as pl
from jax.experimental.pallas import tpu as pltpu
from jax.experimental.pallas import tpu_sc as plsc

N_NODES_K = 10000
N_EDGES_K = 320000
NC = 2
NS = 16
NW = NC * NS
CH = 128
NCH_TOT = N_EDGES_K // CH
BASE_CH = NCH_TOT // NW
EXTRA_W = NCH_TOT - BASE_CH * NW
STRIPE = 640
SUB = 80
DEG_W = 16
PSTRIPE = 320
PSUB = 40

_mesh = plsc.VectorSubcoreMesh(core_axis_name="c", subcore_axis_name="s")
_sc_params = pltpu.CompilerParams(use_tc_tiling_on_sc=False)


def _zero_rows(ref, nrows, width):
    def body(i, _):
        for k in range(width // 16):
            ref[i, pl.ds(k * 16, 16)] = jnp.zeros((16,), jnp.float32)
        return 0

    lax.fori_loop(0, nrows, body, 0)


def _load_worker_indices(idx2_hbm, idx_v, wid):
    base = wid * BASE_CH
    pltpu.sync_copy(idx2_hbm.at[pl.ds(base, BASE_CH)],
                    idx_v.at[pl.ds(0, BASE_CH)])

    @pl.when(wid < EXTRA_W)
    def _():
        pltpu.sync_copy(idx2_hbm.at[NW * BASE_CH + wid], idx_v.at[BASE_CH])


@functools.partial(
    pl.kernel,
    out_type=jax.ShapeDtypeStruct((NC, N_NODES_K, DEG_W), jnp.float32),
    mesh=_mesh,
    scratch_types=[
        pltpu.VMEM((BASE_CH + 1, CH), jnp.int32),
        pltpu.VMEM((CH, DEG_W), jnp.float32),
        pltpu.VMEM((SUB, DEG_W), jnp.float32),
        pltpu.VMEM_SHARED((N_NODES_K, DEG_W), jnp.float32),
    ],
    compiler_params=_sc_params,
)
def _deg_partials(dst2_hbm, out_hbm, dst_v, ones_v, zbuf_v, acc_sh):
    cid = lax.axis_index("c")
    tid = lax.axis_index("s")
    wid = tid * NC + cid

    def fill(i, _):
        ones_v[i, pl.ds(0, 16)] = jnp.full((16,), 1.0, jnp.float32)
        return 0

    lax.fori_loop(0, CH, fill, 0)
    _zero_rows(zbuf_v, SUB, DEG_W)

    row0 = tid * STRIPE
    nsub = jnp.where(tid == NS - 1,
                     (N_NODES_K - (NS - 1) * STRIPE) // SUB, STRIPE // SUB)

    def zero_sub(k, _):
        r = pl.multiple_of(row0 + k * SUB, 8)
        pltpu.sync_copy(zbuf_v, acc_sh.at[pl.ds(r, SUB)])
        return 0

    lax.fori_loop(0, nsub, zero_sub, 0)

    _load_worker_indices(dst2_hbm, dst_v, wid)

    plsc.subcore_barrier()

    def chunk(j, _):
        pltpu.sync_copy(ones_v, acc_sh.at[dst_v.at[j]], add=True)
        return 0

    lax.fori_loop(0, BASE_CH, chunk, 0)

    @pl.when(wid < EXTRA_W)
    def _():
        pltpu.sync_copy(ones_v, acc_sh.at[dst_v.at[BASE_CH]], add=True)

    plsc.subcore_barrier()

    def readback(k, _):
        r = pl.multiple_of(row0 + k * SUB, 8)
        pltpu.sync_copy(acc_sh.at[pl.ds(r, SUB)], zbuf_v)
        pltpu.sync_copy(zbuf_v, out_hbm.at[cid, pl.ds(r, SUB)])
        return 0

    lax.fori_loop(0, nsub, readback, 0)


@functools.partial(
    pl.kernel,
    out_type=jax.ShapeDtypeStruct((NC, N_NODES_K, 64), jnp.float32),
    mesh=_mesh,
    scratch_types=[
        pltpu.VMEM((BASE_CH + 1, CH), jnp.int32),
        pltpu.VMEM((BASE_CH + 1, CH), jnp.int32),
        pltpu.VMEM((2, CH, 64), jnp.float32),
        pltpu.VMEM((SUB, 64), jnp.float32),
        pltpu.VMEM_SHARED((N_NODES_K, 64), jnp.float32),
        pltpu.SemaphoreType.DMA,
    ],
    compiler_params=_sc_params,
)
def _msg_partials(htp_hbm, src2_hbm, dst2_hbm, out_hbm,
                  src_v, dst_v, rows_v, zbuf_v, acc_sh, sem):
    cid = lax.axis_index("c")
    tid = lax.axis_index("s")
    wid = tid * NC + cid

    htv = htp_hbm
    outv = out_hbm.at[cid]

    _zero_rows(zbuf_v, SUB, 64)

    row0 = tid * STRIPE
    nsub = jnp.where(tid == NS - 1,
                     (N_NODES_K - (NS - 1) * STRIPE) // SUB, STRIPE // SUB)

    def zero_sub(k, _):
        r = pl.multiple_of(row0 + k * SUB, 8)
        pltpu.sync_copy(zbuf_v, acc_sh.at[pl.ds(r, SUB)])
        return 0

    lax.fori_loop(0, nsub, zero_sub, 0)

    _load_worker_indices(src2_hbm, src_v, wid)
    _load_worker_indices(dst2_hbm, dst_v, wid)

    plsc.subcore_barrier()

    pltpu.async_copy(htv.at[src_v.at[0]], rows_v.at[0], sem)
    pltpu.async_copy(htv.at[src_v.at[1]], rows_v.at[1], sem)

    def group(g, _):
        j0 = 2 * g
        for b in range(2):
            j = j0 + b
            pltpu.make_async_copy(
                htv.at[src_v.at[0]], rows_v.at[b], sem).wait()
            pltpu.sync_copy(rows_v.at[b], acc_sh.at[dst_v.at[j]], add=True)

            @pl.when(j + 2 < BASE_CH)
            def _():
                pltpu.async_copy(htv.at[src_v.at[j + 2]], rows_v.at[b], sem)
        return 0

    lax.fori_loop(0, BASE_CH // 2, group, 0)

    @pl.when(wid < EXTRA_W)
    def _():
        pltpu.sync_copy(htv.at[src_v.at[BASE_CH]], rows_v.at[0])
        pltpu.sync_copy(rows_v.at[0], acc_sh.at[dst_v.at[BASE_CH]], add=True)

    plsc.subcore_barrier()

    def readback(k, _):
        r = pl.multiple_of(row0 + k * SUB, 8)
        pltpu.sync_copy(acc_sh.at[pl.ds(r, SUB)], zbuf_v)
        pltpu.sync_copy(zbuf_v, outv.at[pl.ds(r, SUB)])
        return 0

    lax.fori_loop(0, nsub, readback, 0)


def _mm1_body(x_ref, w1_ref, h_ref):
    h_ref[...] = jnp.dot(x_ref[...], w1_ref[...],
                         preferred_element_type=jnp.float32)


def _scale1_body(degp_ref, h_ref, ht_ref, dinv_ref):
    degp = degp_ref[...]
    deg = degp[0, :, 0] + degp[1, :, 0] + 1.0
    dinv = lax.rsqrt(deg)
    ht_ref[...] = h_ref[...] * dinv[:, None]
    dinv_ref[...] = dinv[:, None]


def _tc2_body(rv_ref, ht_ref, dinv_ref, b1_ref, w2_ref, ht2_ref):
    rv = rv_ref[...]
    t = rv[0] + rv[1] + ht_ref[...]
    dinv = dinv_ref[...]
    out1 = dinv * t + b1_ref[...]
    h1 = jnp.maximum(out1, 0.0)
    h2 = jnp.dot(h1, w2_ref[...], preferred_element_type=jnp.float32)
    ht2_ref[...] = h2 * dinv


def _tc3_body(rv_ref, ht2_ref, dinv_ref, b2_ref, out_ref):
    rv = rv_ref[...]
    t = rv[0] + rv[1] + ht2_ref[...]
    out_ref[...] = dinv_ref[...] * t + b2_ref[...]


@jax.jit
def kernel(x, edge_index, W1, b1, W2, b2):
    n = N_NODES_K
    src2 = edge_index[0].reshape(NCH_TOT, CH)
    dst2 = edge_index[1].reshape(NCH_TOT, CH)

    degp = _deg_partials(dst2)

    h = pl.pallas_call(
        _mm1_body,
        grid=(5,),
        in_specs=[
            pl.BlockSpec((n // 5, 128), lambda i: (i, 0)),
            pl.BlockSpec((128, 64), lambda i: (0, 0)),
        ],
        out_specs=pl.BlockSpec((n // 5, 64), lambda i: (i, 0)),
        out_shape=jax.ShapeDtypeStruct((n, 64), jnp.float32),
    )(x, W1)

    ht, dinv = pl.pallas_call(
        _scale1_body,
        out_shape=[
            jax.ShapeDtypeStruct((n, 64), jnp.float32),
            jax.ShapeDtypeStruct((n, 1), jnp.float32),
        ],
    )(degp, h)

    r1 = _msg_partials(ht, src2, dst2)

    ht2 = pl.pallas_call(
        _tc2_body,
        out_shape=jax.ShapeDtypeStruct((n, 64), jnp.float32),
    )(r1, ht, dinv, b1.reshape(1, 64), W2)

    r2 = _msg_partials(ht2, src2, dst2)

    out = pl.pallas_call(
        _tc3_body,
        out_shape=jax.ShapeDtypeStruct((n, 64), jnp.float32),
    )(r2, ht2, dinv, b2.reshape(1, 64))

    return out


# speedup vs baseline: 1.1368x; 1.1368x over previous
import functools

import jax
import jax.numpy as jnp
from jax import lax
from jax.experimental import pallas as pl
from jax.experimental.pallas import tpu as pltpu
from jax.experimental.pallas import tpu_sc as plsc

N_NODES_K = 10000
N_EDGES_K = 320000
NC = 2
NS = 16
NW = NC * NS
CH = 128
NCH_TOT = N_EDGES_K // CH
BASE_CH = NCH_TOT // NW
EXTRA_W = NCH_TOT - BASE_CH * NW
STRIPE = 640
SUB = 80
DEG_W = 16
NBUF = 8
DEPTH = 4

_mesh = plsc.VectorSubcoreMesh(core_axis_name="c", subcore_axis_name="s")
_sc_params = pltpu.CompilerParams(use_tc_tiling_on_sc=False)


def _zero_rows(ref, nrows, width):
    def body(i, _):
        for k in range(width // 16):
            ref[i, pl.ds(k * 16, 16)] = jnp.zeros((16,), jnp.float32)
        return 0

    lax.fori_loop(0, nrows, body, 0)


def _load_worker_indices(idx2_hbm, idx_v, wid):
    base = wid * BASE_CH
    pltpu.sync_copy(idx2_hbm.at[pl.ds(base, BASE_CH)],
                    idx_v.at[pl.ds(0, BASE_CH)])

    @pl.when(wid < EXTRA_W)
    def _():
        pltpu.sync_copy(idx2_hbm.at[NW * BASE_CH + wid], idx_v.at[BASE_CH])


@functools.partial(
    pl.kernel,
    out_type=jax.ShapeDtypeStruct((NC, N_NODES_K, DEG_W), jnp.float32),
    mesh=_mesh,
    scratch_types=[
        pltpu.VMEM((BASE_CH + 1, CH), jnp.int32),
        pltpu.VMEM((CH, DEG_W), jnp.float32),
        pltpu.VMEM((SUB, DEG_W), jnp.float32),
        pltpu.VMEM_SHARED((N_NODES_K, DEG_W), jnp.float32),
    ],
    compiler_params=_sc_params,
)
def _deg_partials(dst2_hbm, out_hbm, dst_v, ones_v, zbuf_v, acc_sh):
    cid = lax.axis_index("c")
    tid = lax.axis_index("s")
    wid = tid * NC + cid

    def fill(i, _):
        ones_v[i, pl.ds(0, 16)] = jnp.full((16,), 1.0, jnp.float32)
        return 0

    lax.fori_loop(0, CH, fill, 0)
    _zero_rows(zbuf_v, SUB, DEG_W)

    row0 = tid * STRIPE
    nsub = jnp.where(tid == NS - 1,
                     (N_NODES_K - (NS - 1) * STRIPE) // SUB, STRIPE // SUB)

    def zero_sub(k, _):
        r = pl.multiple_of(row0 + k * SUB, 8)
        pltpu.sync_copy(zbuf_v, acc_sh.at[pl.ds(r, SUB)])
        return 0

    lax.fori_loop(0, nsub, zero_sub, 0)

    _load_worker_indices(dst2_hbm, dst_v, wid)

    plsc.subcore_barrier()

    def chunk(j, _):
        pltpu.sync_copy(ones_v, acc_sh.at[dst_v.at[j]], add=True)
        return 0

    lax.fori_loop(0, BASE_CH, chunk, 0)

    @pl.when(wid < EXTRA_W)
    def _():
        pltpu.sync_copy(ones_v, acc_sh.at[dst_v.at[BASE_CH]], add=True)

    plsc.subcore_barrier()

    def readback(k, _):
        r = pl.multiple_of(row0 + k * SUB, 8)
        pltpu.sync_copy(acc_sh.at[pl.ds(r, SUB)], zbuf_v)
        pltpu.sync_copy(zbuf_v, out_hbm.at[cid, pl.ds(r, SUB)])
        return 0

    lax.fori_loop(0, nsub, readback, 0)


@functools.partial(
    pl.kernel,
    out_type=jax.ShapeDtypeStruct((NC, N_NODES_K, 64), jnp.float32),
    mesh=_mesh,
    scratch_types=[
        pltpu.VMEM((BASE_CH + 1, CH), jnp.int32),
        pltpu.VMEM((BASE_CH + 1, CH), jnp.int32),
        pltpu.VMEM((NBUF, CH, 64), jnp.float32),
        pltpu.VMEM((SUB, 64), jnp.float32),
        pltpu.VMEM_SHARED((N_NODES_K, 64), jnp.float32),
        pltpu.SemaphoreType.DMA,
        pltpu.SemaphoreType.DMA,
    ],
    compiler_params=_sc_params,
)
def _msg_partials(htp_hbm, src2_hbm, dst2_hbm, out_hbm,
                  src_v, dst_v, rows_v, zbuf_v, acc_sh, sem_g, sem_s):
    cid = lax.axis_index("c")
    tid = lax.axis_index("s")
    wid = tid * NC + cid

    htv = htp_hbm
    outv = out_hbm.at[cid]

    _zero_rows(zbuf_v, SUB, 64)

    row0 = tid * STRIPE
    nsub = jnp.where(tid == NS - 1,
                     (N_NODES_K - (NS - 1) * STRIPE) // SUB, STRIPE // SUB)

    def zero_sub(k, _):
        r = pl.multiple_of(row0 + k * SUB, 8)
        pltpu.sync_copy(zbuf_v, acc_sh.at[pl.ds(r, SUB)])
        return 0

    lax.fori_loop(0, nsub, zero_sub, 0)

    _load_worker_indices(src2_hbm, src_v, wid)
    _load_worker_indices(dst2_hbm, dst_v, wid)

    plsc.subcore_barrier()

    for b in range(DEPTH):
        pltpu.async_copy(htv.at[src_v.at[b]], rows_v.at[b], sem_g)

    def group(g, _):
        j0 = NBUF * g
        for b in range(NBUF):
            j = j0 + b

            @pl.when(j < BASE_CH)
            def _():
                pltpu.make_async_copy(
                    htv.at[src_v.at[0]], rows_v.at[b], sem_g).wait()
                pltpu.async_copy(rows_v.at[b], acc_sh.at[dst_v.at[j]],
                                 sem_s, add=True)

                @pl.when(j >= DEPTH)
                def _():
                    pltpu.make_async_copy(
                        rows_v.at[0], acc_sh.at[dst_v.at[0]], sem_s).wait()

                @pl.when(j + DEPTH < BASE_CH)
                def _():
                    pltpu.async_copy(htv.at[src_v.at[j + DEPTH]],
                                     rows_v.at[(b + DEPTH) % NBUF], sem_g)
        return 0

    lax.fori_loop(0, (BASE_CH + NBUF - 1) // NBUF, group, 0)

    for _ in range(DEPTH):
        pltpu.make_async_copy(
            rows_v.at[0], acc_sh.at[dst_v.at[0]], sem_s).wait()

    @pl.when(wid < EXTRA_W)
    def _():
        pltpu.sync_copy(htv.at[src_v.at[BASE_CH]], rows_v.at[0])
        pltpu.sync_copy(rows_v.at[0], acc_sh.at[dst_v.at[BASE_CH]], add=True)

    plsc.subcore_barrier()

    def readback(k, _):
        r = pl.multiple_of(row0 + k * SUB, 8)
        pltpu.sync_copy(acc_sh.at[pl.ds(r, SUB)], zbuf_v)
        pltpu.sync_copy(zbuf_v, outv.at[pl.ds(r, SUB)])
        return 0

    lax.fori_loop(0, nsub, readback, 0)


def _mm1_body(x_ref, w1_ref, h_ref):
    h_ref[...] = jnp.dot(x_ref[...], w1_ref[...],
                         preferred_element_type=jnp.float32)


def _scale1_body(degp_ref, h_ref, ht_ref, dinv_ref):
    degp = degp_ref[...]
    deg = degp[0, :, 0] + degp[1, :, 0] + 1.0
    dinv = lax.rsqrt(deg)
    ht_ref[...] = h_ref[...] * dinv[:, None]
    dinv_ref[...] = dinv[:, None]


def _tc2_body(rv_ref, ht_ref, dinv_ref, b1_ref, w2_ref, ht2_ref):
    rv = rv_ref[...]
    t = rv[0] + rv[1] + ht_ref[...]
    dinv = dinv_ref[...]
    out1 = dinv * t + b1_ref[...]
    h1 = jnp.maximum(out1, 0.0)
    h2 = jnp.dot(h1, w2_ref[...], preferred_element_type=jnp.float32)
    ht2_ref[...] = h2 * dinv


def _tc3_body(rv_ref, ht2_ref, dinv_ref, b2_ref, out_ref):
    rv = rv_ref[...]
    t = rv[0] + rv[1] + ht2_ref[...]
    out_ref[...] = dinv_ref[...] * t + b2_ref[...]


@jax.jit
def kernel(x, edge_index, W1, b1, W2, b2):
    n = N_NODES_K
    src2 = edge_index[0].reshape(NCH_TOT, CH)
    dst2 = edge_index[1].reshape(NCH_TOT, CH)

    degp = _deg_partials(dst2)

    h = pl.pallas_call(
        _mm1_body,
        grid=(5,),
        in_specs=[
            pl.BlockSpec((n // 5, 128), lambda i: (i, 0)),
            pl.BlockSpec((128, 64), lambda i: (0, 0)),
        ],
        out_specs=pl.BlockSpec((n // 5, 64), lambda i: (i, 0)),
        out_shape=jax.ShapeDtypeStruct((n, 64), jnp.float32),
    )(x, W1)

    ht, dinv = pl.pallas_call(
        _scale1_body,
        out_shape=[
            jax.ShapeDtypeStruct((n, 64), jnp.float32),
            jax.ShapeDtypeStruct((n, 1), jnp.float32),
        ],
    )(degp, h)

    r1 = _msg_partials(ht, src2, dst2)

    ht2 = pl.pallas_call(
        _tc2_body,
        out_shape=jax.ShapeDtypeStruct((n, 64), jnp.float32),
    )(r1, ht, dinv, b1.reshape(1, 64), W2)

    r2 = _msg_partials(ht2, src2, dst2)

    out = pl.pallas_call(
        _tc3_body,
        out_shape=jax.ShapeDtypeStruct((n, 64), jnp.float32),
    )(r2, ht2, dinv, b2.reshape(1, 64))

    return out


# speedup vs baseline: 1.1568x; 1.0176x over previous
import functools

import jax
import jax.numpy as jnp
from jax import lax
from jax.experimental import pallas as pl
from jax.experimental.pallas import tpu as pltpu
from jax.experimental.pallas import tpu_sc as plsc

N_NODES_K = 10000
N_EDGES_K = 320000
NC = 2
NS = 16
NW = NC * NS
CH = 128
NCH_TOT = N_EDGES_K // CH
BASE_CH = NCH_TOT // NW
EXTRA_W = NCH_TOT - BASE_CH * NW
STRIPE = 640
SUB = 80
DEG_W = 16
NBUF = 8
DEPTH = 4
DEG_DEPTH = 16

_mesh = plsc.VectorSubcoreMesh(core_axis_name="c", subcore_axis_name="s")
_sc_params = pltpu.CompilerParams(use_tc_tiling_on_sc=False)


def _zero_rows(ref, nrows, width):
    def body(i, _):
        for k in range(width // 16):
            ref[i, pl.ds(k * 16, 16)] = jnp.zeros((16,), jnp.float32)
        return 0

    lax.fori_loop(0, nrows, body, 0)


def _load_worker_indices(idx2_hbm, idx_v, wid):
    base = wid * BASE_CH
    pltpu.sync_copy(idx2_hbm.at[pl.ds(base, BASE_CH)],
                    idx_v.at[pl.ds(0, BASE_CH)])

    @pl.when(wid < EXTRA_W)
    def _():
        pltpu.sync_copy(idx2_hbm.at[NW * BASE_CH + wid], idx_v.at[BASE_CH])


@functools.partial(
    pl.kernel,
    out_type=jax.ShapeDtypeStruct((NC, N_NODES_K, DEG_W), jnp.float32),
    mesh=_mesh,
    scratch_types=[
        pltpu.VMEM((BASE_CH + 1, CH), jnp.int32),
        pltpu.VMEM((CH, DEG_W), jnp.float32),
        pltpu.VMEM((SUB, DEG_W), jnp.float32),
        pltpu.VMEM_SHARED((N_NODES_K, DEG_W), jnp.float32),
        pltpu.SemaphoreType.DMA,
    ],
    compiler_params=_sc_params,
)
def _deg_partials(dst2_hbm, out_hbm, dst_v, ones_v, zbuf_v, acc_sh, sem):
    cid = lax.axis_index("c")
    tid = lax.axis_index("s")
    wid = tid * NC + cid

    def fill(i, _):
        ones_v[i, pl.ds(0, 16)] = jnp.full((16,), 1.0, jnp.float32)
        return 0

    lax.fori_loop(0, CH, fill, 0)
    _zero_rows(zbuf_v, SUB, DEG_W)

    row0 = tid * STRIPE
    nsub = jnp.where(tid == NS - 1,
                     (N_NODES_K - (NS - 1) * STRIPE) // SUB, STRIPE // SUB)

    def zero_sub(k, _):
        r = pl.multiple_of(row0 + k * SUB, 8)
        pltpu.sync_copy(zbuf_v, acc_sh.at[pl.ds(r, SUB)])
        return 0

    lax.fori_loop(0, nsub, zero_sub, 0)

    _load_worker_indices(dst2_hbm, dst_v, wid)

    plsc.subcore_barrier()

    def chunk(j, _):
        pltpu.async_copy(ones_v, acc_sh.at[dst_v.at[j]], sem, add=True)

        @pl.when(j >= DEG_DEPTH)
        def _():
            pltpu.make_async_copy(
                ones_v, acc_sh.at[dst_v.at[0]], sem).wait()
        return 0

    lax.fori_loop(0, BASE_CH, chunk, 0)

    @pl.when(wid < EXTRA_W)
    def _():
        pltpu.async_copy(ones_v, acc_sh.at[dst_v.at[BASE_CH]], sem, add=True)

    def drain(j, _):
        pltpu.make_async_copy(ones_v, acc_sh.at[dst_v.at[0]], sem).wait()
        return 0

    lax.fori_loop(0, DEG_DEPTH, drain, 0)

    @pl.when(wid < EXTRA_W)
    def _():
        pltpu.make_async_copy(ones_v, acc_sh.at[dst_v.at[0]], sem).wait()

    plsc.subcore_barrier()

    def readback(k, _):
        r = pl.multiple_of(row0 + k * SUB, 8)
        pltpu.sync_copy(acc_sh.at[pl.ds(r, SUB)], zbuf_v)
        pltpu.sync_copy(zbuf_v, out_hbm.at[cid, pl.ds(r, SUB)])
        return 0

    lax.fori_loop(0, nsub, readback, 0)


@functools.partial(
    pl.kernel,
    out_type=jax.ShapeDtypeStruct((NC, N_NODES_K, 64), jnp.float32),
    mesh=_mesh,
    scratch_types=[
        pltpu.VMEM((BASE_CH + 1, CH), jnp.int32),
        pltpu.VMEM((BASE_CH + 1, CH), jnp.int32),
        pltpu.VMEM((NBUF, CH, 64), jnp.float32),
        pltpu.VMEM((SUB, 64), jnp.float32),
        pltpu.VMEM_SHARED((N_NODES_K, 64), jnp.float32),
        pltpu.SemaphoreType.DMA,
        pltpu.SemaphoreType.DMA,
    ],
    compiler_params=_sc_params,
)
def _msg_partials(htp_hbm, src2_hbm, dst2_hbm, out_hbm,
                  src_v, dst_v, rows_v, zbuf_v, acc_sh, sem_g, sem_s):
    cid = lax.axis_index("c")
    tid = lax.axis_index("s")
    wid = tid * NC + cid

    htv = htp_hbm
    outv = out_hbm.at[cid]

    _zero_rows(zbuf_v, SUB, 64)

    row0 = tid * STRIPE
    nsub = jnp.where(tid == NS - 1,
                     (N_NODES_K - (NS - 1) * STRIPE) // SUB, STRIPE // SUB)

    def zero_sub(k, _):
        r = pl.multiple_of(row0 + k * SUB, 8)
        pltpu.sync_copy(zbuf_v, acc_sh.at[pl.ds(r, SUB)])
        return 0

    lax.fori_loop(0, nsub, zero_sub, 0)

    _load_worker_indices(src2_hbm, src_v, wid)
    _load_worker_indices(dst2_hbm, dst_v, wid)

    plsc.subcore_barrier()

    for b in range(DEPTH):
        pltpu.async_copy(htv.at[src_v.at[b]], rows_v.at[b], sem_g)

    def group(g, _):
        j0 = NBUF * g
        for b in range(NBUF):
            j = j0 + b

            @pl.when(j < BASE_CH)
            def _():
                pltpu.make_async_copy(
                    htv.at[src_v.at[0]], rows_v.at[b], sem_g).wait()
                pltpu.async_copy(rows_v.at[b], acc_sh.at[dst_v.at[j]],
                                 sem_s, add=True)

                @pl.when(j >= DEPTH)
                def _():
                    pltpu.make_async_copy(
                        rows_v.at[0], acc_sh.at[dst_v.at[0]], sem_s).wait()

                @pl.when(j + DEPTH < BASE_CH)
                def _():
                    pltpu.async_copy(htv.at[src_v.at[j + DEPTH]],
                                     rows_v.at[(b + DEPTH) % NBUF], sem_g)
        return 0

    lax.fori_loop(0, (BASE_CH + NBUF - 1) // NBUF, group, 0)

    for _ in range(DEPTH):
        pltpu.make_async_copy(
            rows_v.at[0], acc_sh.at[dst_v.at[0]], sem_s).wait()

    @pl.when(wid < EXTRA_W)
    def _():
        pltpu.sync_copy(htv.at[src_v.at[BASE_CH]], rows_v.at[0])
        pltpu.sync_copy(rows_v.at[0], acc_sh.at[dst_v.at[BASE_CH]], add=True)

    plsc.subcore_barrier()

    def readback(k, _):
        r = pl.multiple_of(row0 + k * SUB, 8)
        pltpu.sync_copy(acc_sh.at[pl.ds(r, SUB)], zbuf_v)
        pltpu.sync_copy(zbuf_v, outv.at[pl.ds(r, SUB)])
        return 0

    lax.fori_loop(0, nsub, readback, 0)


def _mm1_body(x_ref, w1_ref, h_ref):
    h_ref[...] = jnp.dot(x_ref[...], w1_ref[...],
                         preferred_element_type=jnp.float32)


def _scale1_body(degp_ref, h_ref, ht_ref, dinv_ref):
    degp = degp_ref[...]
    deg = degp[0, :, 0] + degp[1, :, 0] + 1.0
    dinv = lax.rsqrt(deg)
    ht_ref[...] = h_ref[...] * dinv[:, None]
    dinv_ref[...] = dinv[:, None]


def _tc2_body(rv_ref, ht_ref, dinv_ref, b1_ref, w2_ref, ht2_ref):
    rv = rv_ref[...]
    t = rv[0] + rv[1] + ht_ref[...]
    dinv = dinv_ref[...]
    out1 = dinv * t + b1_ref[...]
    h1 = jnp.maximum(out1, 0.0)
    h2 = jnp.dot(h1, w2_ref[...], preferred_element_type=jnp.float32)
    ht2_ref[...] = h2 * dinv


def _tc3_body(rv_ref, ht2_ref, dinv_ref, b2_ref, out_ref):
    rv = rv_ref[...]
    t = rv[0] + rv[1] + ht2_ref[...]
    out_ref[...] = dinv_ref[...] * t + b2_ref[...]


@jax.jit
def kernel(x, edge_index, W1, b1, W2, b2):
    n = N_NODES_K
    src2 = edge_index[0].reshape(NCH_TOT, CH)
    dst2 = edge_index[1].reshape(NCH_TOT, CH)

    degp = _deg_partials(dst2)

    h = pl.pallas_call(
        _mm1_body,
        grid=(5,),
        in_specs=[
            pl.BlockSpec((n // 5, 128), lambda i: (i, 0)),
            pl.BlockSpec((128, 64), lambda i: (0, 0)),
        ],
        out_specs=pl.BlockSpec((n // 5, 64), lambda i: (i, 0)),
        out_shape=jax.ShapeDtypeStruct((n, 64), jnp.float32),
    )(x, W1)

    ht, dinv = pl.pallas_call(
        _scale1_body,
        out_shape=[
            jax.ShapeDtypeStruct((n, 64), jnp.float32),
            jax.ShapeDtypeStruct((n, 1), jnp.float32),
        ],
    )(degp, h)

    r1 = _msg_partials(ht, src2, dst2)

    ht2 = pl.pallas_call(
        _tc2_body,
        out_shape=jax.ShapeDtypeStruct((n, 64), jnp.float32),
    )(r1, ht, dinv, b1.reshape(1, 64), W2)

    r2 = _msg_partials(ht2, src2, dst2)

    out = pl.pallas_call(
        _tc3_body,
        out_shape=jax.ShapeDtypeStruct((n, 64), jnp.float32),
    )(r2, ht2, dinv, b2.reshape(1, 64))

    return out


# speedup vs baseline: 1.1598x; 1.0026x over previous
import functools

import jax
import jax.numpy as jnp
from jax import lax
from jax.experimental import pallas as pl
from jax.experimental.pallas import tpu as pltpu
from jax.experimental.pallas import tpu_sc as plsc

N_NODES_K = 10000
N_EDGES_K = 320000
NC = 2
NS = 16
NW = NC * NS
CH = 128
NCH_TOT = N_EDGES_K // CH
BASE_CH = NCH_TOT // NW
EXTRA_W = NCH_TOT - BASE_CH * NW
STRIPE = 640
SUB = 80
DEG_W = 16
NBUF = 8
DEPTH = 4
DEG_DEPTH = 16

_mesh = plsc.VectorSubcoreMesh(core_axis_name="c", subcore_axis_name="s")
_sc_params = pltpu.CompilerParams(use_tc_tiling_on_sc=False)


def _zero_rows(ref, nrows, width):
    def body(i, _):
        for k in range(width // 16):
            ref[i, pl.ds(k * 16, 16)] = jnp.zeros((16,), jnp.float32)
        return 0

    lax.fori_loop(0, nrows, body, 0)


def _load_worker_indices(idx2_hbm, idx_v, wid):
    base = wid * BASE_CH
    pltpu.sync_copy(idx2_hbm.at[pl.ds(base, BASE_CH)],
                    idx_v.at[pl.ds(0, BASE_CH)])

    @pl.when(wid < EXTRA_W)
    def _():
        pltpu.sync_copy(idx2_hbm.at[NW * BASE_CH + wid], idx_v.at[BASE_CH])


@functools.partial(
    pl.kernel,
    out_type=jax.ShapeDtypeStruct((NC, N_NODES_K, DEG_W), jnp.float32),
    mesh=_mesh,
    scratch_types=[
        pltpu.VMEM((BASE_CH + 1, CH), jnp.int32),
        pltpu.VMEM((CH, DEG_W), jnp.float32),
        pltpu.VMEM((SUB, DEG_W), jnp.float32),
        pltpu.VMEM_SHARED((N_NODES_K, DEG_W), jnp.float32),
        pltpu.SemaphoreType.DMA,
    ],
    compiler_params=_sc_params,
)
def _deg_partials(dst2_hbm, out_hbm, dst_v, ones_v, zbuf_v, acc_sh, sem):
    cid = lax.axis_index("c")
    tid = lax.axis_index("s")
    wid = tid * NC + cid

    def fill(i, _):
        ones_v[i, pl.ds(0, 16)] = jnp.full((16,), 1.0, jnp.float32)
        return 0

    lax.fori_loop(0, CH, fill, 0)
    _zero_rows(zbuf_v, SUB, DEG_W)

    row0 = tid * STRIPE
    nsub = jnp.where(tid == NS - 1,
                     (N_NODES_K - (NS - 1) * STRIPE) // SUB, STRIPE // SUB)

    def zero_sub(k, _):
        r = pl.multiple_of(row0 + k * SUB, 8)
        pltpu.sync_copy(zbuf_v, acc_sh.at[pl.ds(r, SUB)])
        return 0

    lax.fori_loop(0, nsub, zero_sub, 0)

    _load_worker_indices(dst2_hbm, dst_v, wid)

    plsc.subcore_barrier()

    def chunk(j, _):
        pltpu.async_copy(ones_v, acc_sh.at[dst_v.at[j]], sem, add=True)

        @pl.when(j >= DEG_DEPTH)
        def _():
            pltpu.make_async_copy(
                ones_v, acc_sh.at[dst_v.at[0]], sem).wait()
        return 0

    lax.fori_loop(0, BASE_CH, chunk, 0)

    @pl.when(wid < EXTRA_W)
    def _():
        pltpu.async_copy(ones_v, acc_sh.at[dst_v.at[BASE_CH]], sem, add=True)

    def drain(j, _):
        pltpu.make_async_copy(ones_v, acc_sh.at[dst_v.at[0]], sem).wait()
        return 0

    lax.fori_loop(0, DEG_DEPTH, drain, 0)

    @pl.when(wid < EXTRA_W)
    def _():
        pltpu.make_async_copy(ones_v, acc_sh.at[dst_v.at[0]], sem).wait()

    plsc.subcore_barrier()

    def readback(k, _):
        r = pl.multiple_of(row0 + k * SUB, 8)
        pltpu.sync_copy(acc_sh.at[pl.ds(r, SUB)], zbuf_v)
        pltpu.sync_copy(zbuf_v, out_hbm.at[cid, pl.ds(r, SUB)])
        return 0

    lax.fori_loop(0, nsub, readback, 0)


@functools.partial(
    pl.kernel,
    out_type=jax.ShapeDtypeStruct((NC, N_NODES_K, 64), jnp.float32),
    mesh=_mesh,
    scratch_types=[
        pltpu.VMEM((BASE_CH + 1, CH), jnp.int32),
        pltpu.VMEM((BASE_CH + 1, CH), jnp.int32),
        pltpu.VMEM((NBUF, CH, 64), jnp.float32),
        pltpu.VMEM((SUB, 64), jnp.float32),
        pltpu.VMEM_SHARED((N_NODES_K, 64), jnp.float32),
        pltpu.SemaphoreType.DMA,
        pltpu.SemaphoreType.DMA,
    ],
    compiler_params=_sc_params,
)
def _msg_partials(htp_hbm, src2_hbm, dst2_hbm, out_hbm,
                  src_v, dst_v, rows_v, zbuf_v, acc_sh, sem_g, sem_s):
    cid = lax.axis_index("c")
    tid = lax.axis_index("s")
    wid = tid * NC + cid

    htv = htp_hbm
    outv = out_hbm.at[cid]

    _zero_rows(zbuf_v, SUB, 64)

    row0 = tid * STRIPE
    nsub = jnp.where(tid == NS - 1,
                     (N_NODES_K - (NS - 1) * STRIPE) // SUB, STRIPE // SUB)

    def zero_sub(k, _):
        r = pl.multiple_of(row0 + k * SUB, 8)
        pltpu.sync_copy(zbuf_v, acc_sh.at[pl.ds(r, SUB)])
        return 0

    lax.fori_loop(0, nsub, zero_sub, 0)

    _load_worker_indices(src2_hbm, src_v, wid)
    _load_worker_indices(dst2_hbm, dst_v, wid)

    plsc.subcore_barrier()

    for b in range(DEPTH):
        pltpu.async_copy(htv.at[src_v.at[b]], rows_v.at[b], sem_g)

    def group(g, _):
        j0 = NBUF * g
        for b in range(NBUF):
            j = j0 + b

            @pl.when(j < BASE_CH)
            def _():
                pltpu.make_async_copy(
                    htv.at[src_v.at[0]], rows_v.at[b], sem_g).wait()
                pltpu.async_copy(rows_v.at[b], acc_sh.at[dst_v.at[j]],
                                 sem_s, add=True)

                @pl.when(j >= DEPTH)
                def _():
                    pltpu.make_async_copy(
                        rows_v.at[0], acc_sh.at[dst_v.at[0]], sem_s).wait()

                @pl.when(j + DEPTH < BASE_CH)
                def _():
                    pltpu.async_copy(htv.at[src_v.at[j + DEPTH]],
                                     rows_v.at[(b + DEPTH) % NBUF], sem_g)
        return 0

    lax.fori_loop(0, (BASE_CH + NBUF - 1) // NBUF, group, 0)

    for _ in range(DEPTH):
        pltpu.make_async_copy(
            rows_v.at[0], acc_sh.at[dst_v.at[0]], sem_s).wait()

    @pl.when(wid < EXTRA_W)
    def _():
        pltpu.sync_copy(htv.at[src_v.at[BASE_CH]], rows_v.at[0])
        pltpu.sync_copy(rows_v.at[0], acc_sh.at[dst_v.at[BASE_CH]], add=True)

    plsc.subcore_barrier()

    def readback(k, _):
        r = pl.multiple_of(row0 + k * SUB, 8)
        pltpu.sync_copy(acc_sh.at[pl.ds(r, SUB)], zbuf_v)
        pltpu.sync_copy(zbuf_v, outv.at[pl.ds(r, SUB)])
        return 0

    lax.fori_loop(0, nsub, readback, 0)


def _mm_scale_body(degp_ref, x_ref, w1_ref, ht_ref, dinv_ref):
    degp = degp_ref[...]
    deg = degp[0, :, 0] + degp[1, :, 0] + 1.0
    dinv = lax.rsqrt(deg)
    h = jnp.dot(x_ref[...], w1_ref[...], preferred_element_type=jnp.float32)
    ht_ref[...] = h * dinv[:, None]
    dinv_ref[...] = dinv[:, None]


def _tc2_body(rv_ref, ht_ref, dinv_ref, b1_ref, w2_ref, ht2_ref):
    rv = rv_ref[...]
    t = rv[0] + rv[1] + ht_ref[...]
    dinv = dinv_ref[...]
    out1 = dinv * t + b1_ref[...]
    h1 = jnp.maximum(out1, 0.0)
    h2 = jnp.dot(h1, w2_ref[...], preferred_element_type=jnp.float32)
    ht2_ref[...] = h2 * dinv


def _tc3_body(rv_ref, ht2_ref, dinv_ref, b2_ref, out_ref):
    rv = rv_ref[...]
    t = rv[0] + rv[1] + ht2_ref[...]
    out_ref[...] = dinv_ref[...] * t + b2_ref[...]


@jax.jit
def kernel(x, edge_index, W1, b1, W2, b2):
    n = N_NODES_K
    src2 = edge_index[0].reshape(NCH_TOT, CH)
    dst2 = edge_index[1].reshape(NCH_TOT, CH)

    degp = _deg_partials(dst2)

    ht, dinv = pl.pallas_call(
        _mm_scale_body,
        out_shape=[
            jax.ShapeDtypeStruct((n, 64), jnp.float32),
            jax.ShapeDtypeStruct((n, 1), jnp.float32),
        ],
    )(degp, x, W1)

    r1 = _msg_partials(ht, src2, dst2)

    ht2 = pl.pallas_call(
        _tc2_body,
        out_shape=jax.ShapeDtypeStruct((n, 64), jnp.float32),
    )(r1, ht, dinv, b1.reshape(1, 64), W2)

    r2 = _msg_partials(ht2, src2, dst2)

    out = pl.pallas_call(
        _tc3_body,
        out_shape=jax.ShapeDtypeStruct((n, 64), jnp.float32),
    )(r2, ht2, dinv, b2.reshape(1, 64))

    return out


# speedup vs baseline: 1.1941x; 1.0296x over previous
import functools

import jax
import jax.numpy as jnp
from jax import lax
from jax.experimental import pallas as pl
from jax.experimental.pallas import tpu as pltpu
from jax.experimental.pallas import tpu_sc as plsc

N_NODES_K = 10000
N_EDGES_K = 320000
NC = 2
NS = 16
NW = NC * NS
CH = 128
NCH_TOT = N_EDGES_K // CH
BASE_CH = NCH_TOT // NW
EXTRA_W = NCH_TOT - BASE_CH * NW
STRIPE = 640
LAST_STRIPE = N_NODES_K - (NS - 1) * STRIPE
SUB = 80
DEG_W = 16
NBUF = 8
DEPTH = 4
DEG_DEPTH = 16

_mesh = plsc.VectorSubcoreMesh(core_axis_name="c", subcore_axis_name="s")
_sc_params = pltpu.CompilerParams(use_tc_tiling_on_sc=False)


def _zero_rows(ref, nrows, width):
    def body(i, _):
        for k in range(width // 16):
            ref[i, pl.ds(k * 16, 16)] = jnp.zeros((16,), jnp.float32)
        return 0

    lax.fori_loop(0, nrows, body, 0)


def _load_worker_indices(idx2_hbm, idx_v, wid):
    base = wid * BASE_CH
    pltpu.sync_copy(idx2_hbm.at[pl.ds(base, BASE_CH)],
                    idx_v.at[pl.ds(0, BASE_CH)])

    @pl.when(wid < EXTRA_W)
    def _():
        pltpu.sync_copy(idx2_hbm.at[NW * BASE_CH + wid], idx_v.at[BASE_CH])


@functools.partial(
    pl.kernel,
    out_type=jax.ShapeDtypeStruct((NC, N_NODES_K, DEG_W), jnp.float32),
    mesh=_mesh,
    scratch_types=[
        pltpu.VMEM((BASE_CH + 1, CH), jnp.int32),
        pltpu.VMEM((CH, DEG_W), jnp.float32),
        pltpu.VMEM((SUB, DEG_W), jnp.float32),
        pltpu.VMEM_SHARED((N_NODES_K, DEG_W), jnp.float32),
        pltpu.SemaphoreType.DMA,
    ],
    compiler_params=_sc_params,
)
def _deg_partials(dst2_hbm, out_hbm, dst_v, ones_v, zbuf_v, acc_sh, sem):
    cid = lax.axis_index("c")
    tid = lax.axis_index("s")
    wid = tid * NC + cid

    def fill(i, _):
        ones_v[i, pl.ds(0, 16)] = jnp.full((16,), 1.0, jnp.float32)
        return 0

    lax.fori_loop(0, CH, fill, 0)
    _zero_rows(zbuf_v, SUB, DEG_W)

    row0 = tid * STRIPE
    nsub = jnp.where(tid == NS - 1, LAST_STRIPE // SUB, STRIPE // SUB)

    def zero_sub(k, _):
        r = pl.multiple_of(row0 + k * SUB, 8)
        pltpu.async_copy(zbuf_v, acc_sh.at[pl.ds(r, SUB)], sem)
        return 0

    lax.fori_loop(0, nsub, zero_sub, 0)

    _load_worker_indices(dst2_hbm, dst_v, wid)

    def zero_drain(k, _):
        pltpu.make_async_copy(zbuf_v, acc_sh.at[pl.ds(0, SUB)], sem).wait()
        return 0

    lax.fori_loop(0, nsub, zero_drain, 0)

    plsc.subcore_barrier()

    def chunk(j, _):
        pltpu.async_copy(ones_v, acc_sh.at[dst_v.at[j]], sem, add=True)

        @pl.when(j >= DEG_DEPTH)
        def _():
            pltpu.make_async_copy(
                ones_v, acc_sh.at[dst_v.at[0]], sem).wait()
        return 0

    lax.fori_loop(0, BASE_CH, chunk, 0)

    @pl.when(wid < EXTRA_W)
    def _():
        pltpu.async_copy(ones_v, acc_sh.at[dst_v.at[BASE_CH]], sem, add=True)

    def drain(j, _):
        pltpu.make_async_copy(ones_v, acc_sh.at[dst_v.at[0]], sem).wait()
        return 0

    lax.fori_loop(0, DEG_DEPTH, drain, 0)

    @pl.when(wid < EXTRA_W)
    def _():
        pltpu.make_async_copy(ones_v, acc_sh.at[dst_v.at[0]], sem).wait()

    plsc.subcore_barrier()

    @pl.when(tid < NS - 1)
    def _():
        r = pl.multiple_of(row0, 8)
        pltpu.sync_copy(acc_sh.at[pl.ds(r, STRIPE)],
                        out_hbm.at[cid, pl.ds(r, STRIPE)])

    @pl.when(tid == NS - 1)
    def _():
        r = pl.multiple_of(row0, 8)
        pltpu.sync_copy(acc_sh.at[pl.ds(r, LAST_STRIPE)],
                        out_hbm.at[cid, pl.ds(r, LAST_STRIPE)])


@functools.partial(
    pl.kernel,
    out_type=jax.ShapeDtypeStruct((NC, N_NODES_K, 64), jnp.float32),
    mesh=_mesh,
    scratch_types=[
        pltpu.VMEM((BASE_CH + 1, CH), jnp.int32),
        pltpu.VMEM((BASE_CH + 1, CH), jnp.int32),
        pltpu.VMEM((NBUF, CH, 64), jnp.float32),
        pltpu.VMEM((SUB, 64), jnp.float32),
        pltpu.VMEM_SHARED((N_NODES_K, 64), jnp.float32),
        pltpu.SemaphoreType.DMA,
        pltpu.SemaphoreType.DMA,
    ],
    compiler_params=_sc_params,
)
def _msg_partials(htp_hbm, src2_hbm, dst2_hbm, out_hbm,
                  src_v, dst_v, rows_v, zbuf_v, acc_sh, sem_g, sem_s):
    cid = lax.axis_index("c")
    tid = lax.axis_index("s")
    wid = tid * NC + cid

    htv = htp_hbm
    outv = out_hbm.at[cid]

    _zero_rows(zbuf_v, SUB, 64)

    row0 = tid * STRIPE
    nsub = jnp.where(tid == NS - 1, LAST_STRIPE // SUB, STRIPE // SUB)

    def zero_sub(k, _):
        r = pl.multiple_of(row0 + k * SUB, 8)
        pltpu.async_copy(zbuf_v, acc_sh.at[pl.ds(r, SUB)], sem_s)
        return 0

    lax.fori_loop(0, nsub, zero_sub, 0)

    _load_worker_indices(src2_hbm, src_v, wid)
    _load_worker_indices(dst2_hbm, dst_v, wid)

    def zero_drain(k, _):
        pltpu.make_async_copy(zbuf_v, acc_sh.at[pl.ds(0, SUB)], sem_s).wait()
        return 0

    lax.fori_loop(0, nsub, zero_drain, 0)

    plsc.subcore_barrier()

    for b in range(DEPTH):
        pltpu.async_copy(htv.at[src_v.at[b]], rows_v.at[b], sem_g)

    def group(g, _):
        j0 = NBUF * g
        for b in range(NBUF):
            j = j0 + b

            @pl.when(j < BASE_CH)
            def _():
                pltpu.make_async_copy(
                    htv.at[src_v.at[0]], rows_v.at[b], sem_g).wait()
                pltpu.async_copy(rows_v.at[b], acc_sh.at[dst_v.at[j]],
                                 sem_s, add=True)

                @pl.when(j >= DEPTH)
                def _():
                    pltpu.make_async_copy(
                        rows_v.at[0], acc_sh.at[dst_v.at[0]], sem_s).wait()

                @pl.when(j + DEPTH < BASE_CH)
                def _():
                    pltpu.async_copy(htv.at[src_v.at[j + DEPTH]],
                                     rows_v.at[(b + DEPTH) % NBUF], sem_g)
        return 0

    lax.fori_loop(0, (BASE_CH + NBUF - 1) // NBUF, group, 0)

    for _ in range(DEPTH):
        pltpu.make_async_copy(
            rows_v.at[0], acc_sh.at[dst_v.at[0]], sem_s).wait()

    @pl.when(wid < EXTRA_W)
    def _():
        pltpu.sync_copy(htv.at[src_v.at[BASE_CH]], rows_v.at[0])
        pltpu.sync_copy(rows_v.at[0], acc_sh.at[dst_v.at[BASE_CH]], add=True)

    plsc.subcore_barrier()

    @pl.when(tid < NS - 1)
    def _():
        r = pl.multiple_of(row0, 8)
        pltpu.sync_copy(acc_sh.at[pl.ds(r, STRIPE)],
                        outv.at[pl.ds(r, STRIPE)])

    @pl.when(tid == NS - 1)
    def _():
        r = pl.multiple_of(row0, 8)
        pltpu.sync_copy(acc_sh.at[pl.ds(r, LAST_STRIPE)],
                        outv.at[pl.ds(r, LAST_STRIPE)])


def _mm_scale_body(degp_ref, x_ref, w1_ref, ht_ref, dinv_ref):
    degp = degp_ref[...]
    deg = degp[0, :, 0] + degp[1, :, 0] + 1.0
    dinv = lax.rsqrt(deg)
    h = jnp.dot(x_ref[...], w1_ref[...], preferred_element_type=jnp.float32)
    ht_ref[...] = h * dinv[:, None]
    dinv_ref[...] = dinv[:, None]


def _tc2_body(rv_ref, ht_ref, dinv_ref, b1_ref, w2_ref, ht2_ref):
    rv = rv_ref[...]
    t = rv[0] + rv[1] + ht_ref[...]
    dinv = dinv_ref[...]
    out1 = dinv * t + b1_ref[...]
    h1 = jnp.maximum(out1, 0.0)
    h2 = jnp.dot(h1, w2_ref[...], preferred_element_type=jnp.float32)
    ht2_ref[...] = h2 * dinv


def _tc3_body(rv_ref, ht2_ref, dinv_ref, b2_ref, out_ref):
    rv = rv_ref[...]
    t = rv[0] + rv[1] + ht2_ref[...]
    out_ref[...] = dinv_ref[...] * t + b2_ref[...]


@jax.jit
def kernel(x, edge_index, W1, b1, W2, b2):
    n = N_NODES_K
    src2 = edge_index[0].reshape(NCH_TOT, CH)
    dst2 = edge_index[1].reshape(NCH_TOT, CH)

    degp = _deg_partials(dst2)

    ht, dinv = pl.pallas_call(
        _mm_scale_body,
        out_shape=[
            jax.ShapeDtypeStruct((n, 64), jnp.float32),
            jax.ShapeDtypeStruct((n, 1), jnp.float32),
        ],
    )(degp, x, W1)

    r1 = _msg_partials(ht, src2, dst2)

    ht2 = pl.pallas_call(
        _tc2_body,
        out_shape=jax.ShapeDtypeStruct((n, 64), jnp.float32),
    )(r1, ht, dinv, b1.reshape(1, 64), W2)

    r2 = _msg_partials(ht2, src2, dst2)

    out = pl.pallas_call(
        _tc3_body,
        out_shape=jax.ShapeDtypeStruct((n, 64), jnp.float32),
    )(r2, ht2, dinv, b2.reshape(1, 64))

    return out
